# baseline (speedup 1.0000x reference)
import functools

import jax
import jax.numpy as jnp
from jax import lax
from jax.experimental import pallas as pl
from jax.experimental.pallas import tpu as pltpu
from jax.experimental.pallas import tpu_sc as plsc

R, C, K = 16384, 512, 128
NC, NS = 2, 16
NW = NC * NS
ROWS_PER_W = R // NW
CH = 64
NCH = ROWS_PER_W // CH

_mesh = plsc.VectorSubcoreMesh(core_axis_name="c", subcore_axis_name="s")


@functools.partial(
    pl.kernel,
    mesh=_mesh,
    out_type=jax.ShapeDtypeStruct((R, K), jnp.float32),
    scratch_types=[
        pltpu.VMEM((2, CH, C), jnp.float32),
        pltpu.VMEM((2, CH, K), jnp.float32),
        pltpu.SemaphoreType.DMA,
        pltpu.SemaphoreType.DMA,
    ],
    compiler_params=pltpu.CompilerParams(needs_layout_passes=False),
)
def _sc_extract(in_hbm, out_hbm, inbuf, outbuf, in_sem, out_sem):
    wid = lax.axis_index("s") * NC + lax.axis_index("c")
    lane = lax.iota(jnp.int32, 16)
    colv = [lane * 4 + 64 * g for g in range(K // 16)]
    def in_copy(ch):
        return pltpu.make_async_copy(
            in_hbm.at[pl.ds((ch * NW + wid) * CH, CH), :],
            inbuf.at[lax.rem(ch, 2)], in_sem)

    def out_copy(ch):
        return pltpu.make_async_copy(
            outbuf.at[lax.rem(ch, 2)],
            out_hbm.at[pl.ds((ch * NW + wid) * CH, CH), :], out_sem)

    in_copy(0).start()
    in_copy(1).start()

    def chunk_body(ch, _):
        in_copy(ch).wait()

        @pl.when(ch >= 2)
        def _():
            out_copy(ch - 2).wait()

        ib = inbuf.at[lax.rem(ch, 2)]
        ob = outbuf.at[lax.rem(ch, 2)]

        @plsc.parallel_loop(0, CH, unroll=2)
        def body(r, ib=ib, ob=ob, colv=colv):
            rows = jnp.full((16,), r, jnp.int32)
            for g in range(K // 16):
                ob[r, pl.ds(g * 16, 16)] = plsc.load_gather(ib, [rows, colv[g]])
        out_copy(ch).start()

        @pl.when(ch + 2 < NCH)
        def _():
            in_copy(ch + 2).start()

        return 0

    lax.fori_loop(0, NCH, chunk_body, 0, unroll=1)
    out_copy(NCH - 2).wait()
    out_copy(NCH - 1).wait()


def kernel(inputs):
    return _sc_extract(inputs)


# speedup vs baseline: 1.0025x; 1.0025x over previous
import functools

import jax
import jax.numpy as jnp
from jax import lax
from jax.experimental import pallas as pl
from jax.experimental.pallas import tpu as pltpu
from jax.experimental.pallas import tpu_sc as plsc

R, C, K = 16384, 512, 128
NC, NS = 2, 16
NW = NC * NS
ROWS_PER_W = R // NW
CH = 64
NCH = ROWS_PER_W // CH

_mesh = plsc.VectorSubcoreMesh(core_axis_name="c", subcore_axis_name="s")


@functools.partial(
    pl.kernel,
    mesh=_mesh,
    out_type=jax.ShapeDtypeStruct((R, K), jnp.float32),
    scratch_types=[
        pltpu.VMEM((2, CH, C), jnp.float32),
        pltpu.VMEM((2, CH, K), jnp.float32),
        pltpu.SemaphoreType.DMA,
        pltpu.SemaphoreType.DMA,
    ],
    compiler_params=pltpu.CompilerParams(needs_layout_passes=False),
)
def _sc_extract(in_hbm, out_hbm, inbuf, outbuf, in_sem, out_sem):
    wid = lax.axis_index("s") * NC + lax.axis_index("c")
    lane = lax.iota(jnp.int32, 16)
    colv = [lane * 4 + 64 * g for g in range(K // 16)]
    base = wid * ROWS_PER_W

    def in_copy(ch):
        return pltpu.make_async_copy(
            in_hbm.at[pl.ds(base + ch * CH, CH), :],
            inbuf.at[lax.rem(ch, 2)], in_sem)

    def out_copy(ch):
        return pltpu.make_async_copy(
            outbuf.at[lax.rem(ch, 2)],
            out_hbm.at[pl.ds(base + ch * CH, CH), :], out_sem)

    in_copy(0).start()
    in_copy(1).start()

    def chunk_body(ch, _):
        in_copy(ch).wait()

        @pl.when(ch >= 2)
        def _():
            out_copy(ch - 2).wait()

        ib = inbuf.at[lax.rem(ch, 2)]
        ob = outbuf.at[lax.rem(ch, 2)]

        @plsc.parallel_loop(0, CH, unroll=2)
        def body(r, ib=ib, ob=ob, colv=colv):
            rows = jnp.full((16,), r, jnp.int32)
            for g in range(K // 16):
                ob[r, pl.ds(g * 16, 16)] = plsc.load_gather(ib, [rows, colv[g]])
        out_copy(ch).start()

        @pl.when(ch + 2 < NCH)
        def _():
            in_copy(ch + 2).start()

        return 0

    lax.fori_loop(0, NCH, chunk_body, 0, unroll=1)
    out_copy(NCH - 2).wait()
    out_copy(NCH - 1).wait()


def kernel(inputs):
    return _sc_extract(inputs)
